# baseline (speedup 1.0000x reference)
import functools

import jax
import jax.numpy as jnp
from jax import lax
from jax.experimental import pallas as pl
from jax.experimental.pallas import tpu as pltpu
from jax.experimental.pallas import tpu_sc as plsc

_NC = 2
_NS = 16
_NW = _NC * _NS
_L = 16
_CH = 64


def _floor_f32(v):
    t = v.astype(jnp.int32).astype(jnp.float32)
    return jnp.where(v < t, t - 1.0, t)


def _plane_lookup_sc(table, xs, ys, res, C, B):
    b_per_w = B // _NW
    n_chunks = b_per_w // _CH
    assert n_chunks % 2 == 0
    fres = float(res)
    nseg = C // _L

    mesh = plsc.VectorSubcoreMesh(
        core_axis_name="c", subcore_axis_name="s",
        num_cores=_NC, num_subcores=_NS)

    @functools.partial(
        pl.kernel,
        out_type=jax.ShapeDtypeStruct((B, C), jnp.float32),
        mesh=mesh,
        scratch_types=dict(
            xs_v=pltpu.VMEM((_CH,), jnp.float32),
            ys_v=pltpu.VMEM((_CH,), jnp.float32),
            idx0=pltpu.VMEM((4, _CH), jnp.int32),
            idx1=pltpu.VMEM((4, _CH), jnp.int32),
            w0=pltpu.VMEM((4, _CH + _L), jnp.float32),
            w1=pltpu.VMEM((4, _CH + _L), jnp.float32),
            rows0=pltpu.VMEM((4, _CH, C), jnp.float32),
            rows1=pltpu.VMEM((4, _CH, C), jnp.float32),
            out0=pltpu.VMEM((_CH, C), jnp.float32),
            out1=pltpu.VMEM((_CH, C), jnp.float32),
            gsem0=pltpu.SemaphoreType.DMA,
            gsem1=pltpu.SemaphoreType.DMA,
            osem0=pltpu.SemaphoreType.DMA,
            osem1=pltpu.SemaphoreType.DMA,
        ),
    )
    def body(table_hbm, xs_hbm, ys_hbm, out_hbm, *, xs_v, ys_v,
             idx0, idx1, w0, w1, rows0, rows1, out0, out1,
             gsem0, gsem1, osem0, osem1):
        wid = lax.axis_index("s") * _NC + lax.axis_index("c")
        tile_base = wid * b_per_w
        idx = (idx0, idx1)
        wgt = (w0, w1)
        rows = (rows0, rows1)
        outb = (out0, out1)
        gsem = (gsem0, gsem1)
        osem = (osem0, osem1)

        def prep_and_fire(k, b):
            base = tile_base + k * _CH
            pltpu.sync_copy(xs_hbm.at[pl.ds(base, _CH)], xs_v)
            pltpu.sync_copy(ys_hbm.at[pl.ds(base, _CH)], ys_v)
            for i in range(_CH // _L):
                sl = pl.ds(i * _L, _L)
                px = xs_v[sl] * fres - 0.5
                py = ys_v[sl] * fres - 0.5
                x0f = _floor_f32(px)
                y0f = _floor_f32(py)
                wx = px - x0f
                wy = py - y0f
                x0 = jnp.maximum(jnp.minimum(x0f.astype(jnp.int32), res - 1), 0)
                y0 = jnp.maximum(jnp.minimum(y0f.astype(jnp.int32), res - 1), 0)
                x1 = jnp.minimum(x0 + 1, res - 1)
                y1 = jnp.minimum(y0 + 1, res - 1)
                yb0 = y0 * res
                yb1 = y1 * res
                idx[b][0, sl] = yb0 + x0
                idx[b][1, sl] = yb0 + x1
                idx[b][2, sl] = yb1 + x0
                idx[b][3, sl] = yb1 + x1
                omwx = 1.0 - wx
                omwy = 1.0 - wy
                wgt[b][0, sl] = omwy * omwx
                wgt[b][1, sl] = omwy * wx
                wgt[b][2, sl] = wy * omwx
                wgt[b][3, sl] = wy * wx
            for n in range(4):
                pltpu.async_copy(table_hbm.at[idx[b].at[n]], rows[b].at[n],
                                 gsem[b])

        def drain_gathers(b):
            for n in range(4):
                pltpu.make_async_copy(table_hbm.at[idx[b].at[n]],
                                      rows[b].at[n], gsem[b]).wait()

        def wait_out(b):
            pltpu.make_async_copy(outb[b], out_hbm.at[pl.ds(tile_base, _CH)],
                                  osem[b]).wait()

        def blend_and_store(k, b):
            rb = rows[b]
            ob = outb[b]
            wb = wgt[b]

            def blend_body(p, carry):
                a0 = wb[0, pl.ds(p, _L)][0]
                a1 = wb[1, pl.ds(p, _L)][0]
                a2 = wb[2, pl.ds(p, _L)][0]
                a3 = wb[3, pl.ds(p, _L)][0]
                for j in range(nseg):
                    cs = pl.ds(j * _L, _L)
                    ob[p, cs] = (a0 * rb[0, p, cs] + a1 * rb[1, p, cs]
                                 + a2 * rb[2, p, cs] + a3 * rb[3, p, cs])
                return carry

            lax.fori_loop(0, _CH, blend_body, 0)
            base = tile_base + k * _CH
            pltpu.async_copy(ob, out_hbm.at[pl.ds(base, _CH)], osem[b])

        prep_and_fire(0, 0)

        def pair_body(g, carry):
            for b in range(2):
                k = 2 * g + b

                @pl.when(k + 1 < n_chunks)
                def _():
                    prep_and_fire(k + 1, b ^ 1)

                drain_gathers(b)

                @pl.when(k >= 2)
                def _():
                    wait_out(b)

                blend_and_store(k, b)
            return carry

        lax.fori_loop(0, n_chunks // 2, pair_body, 0)
        wait_out(0)
        wait_out(1)

    return body(table, xs, ys)


def kernel(x, mat):
    C = mat.shape[1]
    res = mat.shape[2]
    B = x.shape[0]
    table = jnp.transpose(mat[0], (1, 2, 0)).reshape(res * res, C)
    xs = x[:, 0]
    ys = x[:, 1]
    return _plane_lookup_sc(table, xs, ys, res, C, B)


# speedup vs baseline: 1.2223x; 1.2223x over previous
import functools

import jax
import jax.numpy as jnp
from jax import lax
from jax.experimental import pallas as pl
from jax.experimental.pallas import tpu as pltpu
from jax.experimental.pallas import tpu_sc as plsc

_NC = 2
_NS = 16
_NW = _NC * _NS
_L = 16
_CH = 64


def _floor_f32(v):
    t = v.astype(jnp.int32).astype(jnp.float32)
    return jnp.where(v < t, t - 1.0, t)


def _plane_lookup_sc(table, xs, ys, res, C, B):
    b_per_w = B // _NW
    n_chunks = b_per_w // _CH
    assert n_chunks % 2 == 0
    fres = float(res)
    nseg = C // _L

    mesh = plsc.VectorSubcoreMesh(
        core_axis_name="c", subcore_axis_name="s",
        num_cores=_NC, num_subcores=_NS)

    @functools.partial(
        pl.kernel,
        out_type=jax.ShapeDtypeStruct((B, C), jnp.float32),
        mesh=mesh,
        scratch_types=dict(
            xs_v=pltpu.VMEM((B // _NW,), jnp.float32),
            ys_v=pltpu.VMEM((B // _NW,), jnp.float32),
            idx0=pltpu.VMEM((4, _CH), jnp.int32),
            idx1=pltpu.VMEM((4, _CH), jnp.int32),
            w0=pltpu.VMEM((4, _CH + _L), jnp.float32),
            w1=pltpu.VMEM((4, _CH + _L), jnp.float32),
            rows0=pltpu.VMEM((4, _CH, C), jnp.float32),
            rows1=pltpu.VMEM((4, _CH, C), jnp.float32),
            out0=pltpu.VMEM((_CH, C), jnp.float32),
            out1=pltpu.VMEM((_CH, C), jnp.float32),
            gsem0=pltpu.SemaphoreType.DMA,
            gsem1=pltpu.SemaphoreType.DMA,
            osem0=pltpu.SemaphoreType.DMA,
            osem1=pltpu.SemaphoreType.DMA,
        ),
    )
    def body(table_hbm, xs_hbm, ys_hbm, out_hbm, *, xs_v, ys_v,
             idx0, idx1, w0, w1, rows0, rows1, out0, out1,
             gsem0, gsem1, osem0, osem1):
        wid = lax.axis_index("s") * _NC + lax.axis_index("c")
        tile_base = wid * b_per_w
        idx = (idx0, idx1)
        wgt = (w0, w1)
        rows = (rows0, rows1)
        outb = (out0, out1)
        gsem = (gsem0, gsem1)
        osem = (osem0, osem1)

        pltpu.sync_copy(xs_hbm.at[pl.ds(tile_base, b_per_w)], xs_v)
        pltpu.sync_copy(ys_hbm.at[pl.ds(tile_base, b_per_w)], ys_v)

        def prep_and_fire(k, b):
            for i in range(_CH // _L):
                sl = pl.ds(i * _L, _L)
                cl = pl.ds(k * _CH + i * _L, _L)
                px = xs_v[cl] * fres - 0.5
                py = ys_v[cl] * fres - 0.5
                x0f = _floor_f32(px)
                y0f = _floor_f32(py)
                wx = px - x0f
                wy = py - y0f
                x0 = jnp.maximum(jnp.minimum(x0f.astype(jnp.int32), res - 1), 0)
                y0 = jnp.maximum(jnp.minimum(y0f.astype(jnp.int32), res - 1), 0)
                x1 = jnp.minimum(x0 + 1, res - 1)
                y1 = jnp.minimum(y0 + 1, res - 1)
                yb0 = y0 * res
                yb1 = y1 * res
                idx[b][0, sl] = yb0 + x0
                idx[b][1, sl] = yb0 + x1
                idx[b][2, sl] = yb1 + x0
                idx[b][3, sl] = yb1 + x1
                omwx = 1.0 - wx
                omwy = 1.0 - wy
                wgt[b][0, sl] = omwy * omwx
                wgt[b][1, sl] = omwy * wx
                wgt[b][2, sl] = wy * omwx
                wgt[b][3, sl] = wy * wx
            for n in range(4):
                pltpu.async_copy(table_hbm.at[idx[b].at[n]], rows[b].at[n],
                                 gsem[b])

        def drain_gathers(b):
            for n in range(4):
                pltpu.make_async_copy(table_hbm.at[idx[b].at[n]],
                                      rows[b].at[n], gsem[b]).wait()

        def wait_out(b):
            pltpu.make_async_copy(outb[b], out_hbm.at[pl.ds(tile_base, _CH)],
                                  osem[b]).wait()

        def blend_and_store(k, b):
            rb = rows[b]
            ob = outb[b]
            wb = wgt[b]

            def blend_body(p, carry):
                a0 = wb[0, pl.ds(p, _L)][0]
                a1 = wb[1, pl.ds(p, _L)][0]
                a2 = wb[2, pl.ds(p, _L)][0]
                a3 = wb[3, pl.ds(p, _L)][0]
                for j in range(nseg):
                    cs = pl.ds(j * _L, _L)
                    ob[p, cs] = (a0 * rb[0, p, cs] + a1 * rb[1, p, cs]
                                 + a2 * rb[2, p, cs] + a3 * rb[3, p, cs])
                return carry

            lax.fori_loop(0, _CH, blend_body, 0)
            base = tile_base + k * _CH
            pltpu.async_copy(ob, out_hbm.at[pl.ds(base, _CH)], osem[b])

        prep_and_fire(0, 0)

        def pair_body(g, carry):
            for b in range(2):
                k = 2 * g + b

                @pl.when(k + 1 < n_chunks)
                def _():
                    prep_and_fire(k + 1, b ^ 1)

                drain_gathers(b)

                @pl.when(k >= 2)
                def _():
                    wait_out(b)

                blend_and_store(k, b)
            return carry

        lax.fori_loop(0, n_chunks // 2, pair_body, 0)
        wait_out(0)
        wait_out(1)

    return body(table, xs, ys)


def kernel(x, mat):
    C = mat.shape[1]
    res = mat.shape[2]
    B = x.shape[0]
    table = jnp.transpose(mat[0], (1, 2, 0)).reshape(res * res, C)
    xs = x[:, 0]
    ys = x[:, 1]
    return _plane_lookup_sc(table, xs, ys, res, C, B)
